# baseline (speedup 1.0000x reference)
import jax
import jax.numpy as jnp
from jax.experimental import pallas as pl
from jax.experimental.pallas import tpu as pltpu

_TAP_OF = {(0, 1): 1, (0, 0): 3, (1, 2): 0, (1, 1): 2}


def _make_body(H, W, Cin, Cout):
    HW = H * W
    Hp, Wp = H + 2, W + 2

    def body(x_ref, w_ref, b_ref, o_ref, xpad_ref, pat_ref):
        xt = jnp.transpose(x_ref[0].astype(jnp.bfloat16))

        zr = jnp.zeros((1, Wp, Cin), jnp.bfloat16)
        xpad_ref[0:1] = zr
        xpad_ref[Hp - 1:Hp] = zr
        zc = jnp.zeros((Hp, 1, Cin), jnp.bfloat16)
        xpad_ref[:, 0:1] = zc
        xpad_ref[:, Wp - 1:Wp] = zc
        xpad_ref[1:H + 1, 1:W + 1] = xt.reshape(H, W, Cin)

        for oh in range(3):
            for ow in range(3):
                j = oh * 3 + ow
                pat_ref[:, j * Cin:(j + 1) * Cin] = (
                    xpad_ref[oh:oh + H, ow:ow + W, :].reshape(HW, Cin))

        bias = b_ref[...].astype(jnp.float32)

        streams = []
        for ry in range(2):
            lhs = pat_ref[:, 3 * ry * Cin:(3 * ry + 6) * Cin]
            acc = jnp.dot(lhs, w_ref[ry], preferred_element_type=jnp.float32)
            acc = acc + bias
            y = acc * pl.reciprocal(1.0 + jnp.exp(-acc), approx=True)
            r = jnp.stack([y[:, :Cout], y[:, Cout:]], axis=1)
            streams.append(r.reshape(2 * HW, Cout).reshape(H, 2 * W, Cout))

        ot = jnp.stack(streams, axis=1).reshape(4 * HW, Cout)
        o_ref[0] = jnp.transpose(ot)

    return body


def kernel(x_ncdhw, weight, bias):
    N, Cin, D, H, W = x_ncdhw.shape
    Cout = weight.shape[1]
    HW = H * W

    x3 = x_ncdhw.reshape(N, Cin, D * HW)

    w2 = weight[:, :, 0]
    zero = jnp.zeros((Cin, Cout), w2.dtype)
    wk = jnp.stack([
        jnp.concatenate([
            jnp.concatenate([
                w2[:, :, _TAP_OF[(ry, ry + j2 // 3)], _TAP_OF[(rx, j2 % 3)]]
                if (rx, j2 % 3) in _TAP_OF else zero
                for rx in range(2)], axis=1)
            for j2 in range(6)], axis=0)
        for ry in range(2)], axis=0).astype(jnp.bfloat16)
    b2 = jnp.concatenate([bias, bias]).reshape(1, 2 * Cout).astype(jnp.float32)

    out3 = pl.pallas_call(
        _make_body(H, W, Cin, Cout),
        out_shape=jax.ShapeDtypeStruct((N, Cout, D * 4 * HW), x_ncdhw.dtype),
        grid=(N * D,),
        in_specs=[
            pl.BlockSpec((1, Cin, HW), lambda i: (i // D, 0, i % D)),
            pl.BlockSpec((2, 6 * Cin, 2 * Cout), lambda i: (0, 0, 0)),
            pl.BlockSpec((1, 2 * Cout), lambda i: (0, 0)),
        ],
        out_specs=pl.BlockSpec((1, Cout, 4 * HW), lambda i: (i // D, 0, i % D)),
        scratch_shapes=[
            pltpu.VMEM((H + 2, W + 2, Cin), jnp.bfloat16),
            pltpu.VMEM((HW, 9 * Cin), jnp.bfloat16),
        ],
        compiler_params=pltpu.CompilerParams(
            dimension_semantics=("parallel",),
            vmem_limit_bytes=48 * 1024 * 1024),
    )(x3, wk, b2)

    return out3.reshape(N, Cout, D, 2 * H, 2 * W)


# speedup vs baseline: 1.3799x; 1.3799x over previous
import jax
import jax.numpy as jnp
from jax.experimental import pallas as pl
from jax.experimental.pallas import tpu as pltpu

_TAP_OF = {(0, 1): 1, (0, 0): 3, (1, 2): 0, (1, 1): 2}


def _make_body(H, W, Cin, Cout):
    HW = H * W
    Hp, Wp = H + 2, W + 2

    def body(x_ref, w_ref, b_ref, o_ref, xpad_ref, ot_ref):
        xt = jnp.transpose(x_ref[0].astype(jnp.bfloat16))

        zr = jnp.zeros((1, Wp, Cin), jnp.bfloat16)
        xpad_ref[0:1] = zr
        xpad_ref[Hp - 1:Hp] = zr
        zc = jnp.zeros((Hp, 1, Cin), jnp.bfloat16)
        xpad_ref[:, 0:1] = zc
        xpad_ref[:, Wp - 1:Wp] = zc
        xpad_ref[1:H + 1, 1:W + 1] = xt.reshape(H, W, Cin)

        bias = b_ref[...].astype(jnp.float32)

        for ry in range(2):
            lhs = jnp.concatenate(
                [xpad_ref[oh:oh + H, ow:ow + W, :].reshape(HW, Cin)
                 for oh in (ry, ry + 1) for ow in range(3)], axis=1)
            acc = jnp.dot(lhs, w_ref[ry], preferred_element_type=jnp.float32)
            acc = acc + bias
            y = acc * pl.reciprocal(1.0 + jnp.exp(-acc), approx=True)
            for rx in range(2):
                ot_ref[pl.ds(ry, H, 2), pl.ds(rx, W, 2), :] = (
                    y[:, rx * Cout:(rx + 1) * Cout].reshape(H, W, Cout))

        o_ref[0] = jnp.transpose(ot_ref[...].reshape(4 * HW, Cout))

    return body


def kernel(x_ncdhw, weight, bias):
    N, Cin, D, H, W = x_ncdhw.shape
    Cout = weight.shape[1]
    HW = H * W

    x3 = x_ncdhw.reshape(N, Cin, D * HW)

    w2 = weight[:, :, 0]
    zero = jnp.zeros((Cin, Cout), w2.dtype)
    wk = jnp.stack([
        jnp.concatenate([
            jnp.concatenate([
                w2[:, :, _TAP_OF[(ry, ry + j2 // 3)], _TAP_OF[(rx, j2 % 3)]]
                if (rx, j2 % 3) in _TAP_OF else zero
                for rx in range(2)], axis=1)
            for j2 in range(6)], axis=0)
        for ry in range(2)], axis=0).astype(jnp.bfloat16)
    b2 = jnp.concatenate([bias, bias]).reshape(1, 2 * Cout).astype(jnp.float32)

    out3 = pl.pallas_call(
        _make_body(H, W, Cin, Cout),
        out_shape=jax.ShapeDtypeStruct((N, Cout, D * 4 * HW), x_ncdhw.dtype),
        grid=(N * D,),
        in_specs=[
            pl.BlockSpec((1, Cin, HW), lambda i: (i // D, 0, i % D)),
            pl.BlockSpec((2, 6 * Cin, 2 * Cout), lambda i: (0, 0, 0)),
            pl.BlockSpec((1, 2 * Cout), lambda i: (0, 0)),
        ],
        out_specs=pl.BlockSpec((1, Cout, 4 * HW), lambda i: (i // D, 0, i % D)),
        scratch_shapes=[
            pltpu.VMEM((H + 2, W + 2, Cin), jnp.bfloat16),
            pltpu.VMEM((2 * H, 2 * W, Cout), jnp.float32),
        ],
        compiler_params=pltpu.CompilerParams(
            dimension_semantics=("parallel",),
            vmem_limit_bytes=48 * 1024 * 1024),
    )(x3, wk, b2)

    return out3.reshape(N, Cout, D, 2 * H, 2 * W)


# speedup vs baseline: 1.4916x; 1.0809x over previous
import jax
import jax.numpy as jnp
from jax.experimental import pallas as pl
from jax.experimental.pallas import tpu as pltpu

_TAP_OF = {(0, 1): 1, (0, 0): 3, (1, 2): 0, (1, 1): 2}


def _make_body(H, W, Cin, Cout, TD):
    HW = H * W
    Hp, Wp = H + 2, W + 2

    def body(x_ref, w_ref, b_ref, o_ref, xpad_ref, ot_ref):
        bias = b_ref[...].astype(jnp.float32)

        for d in range(TD):
            xt = jnp.transpose(
                x_ref[0, :, d * HW:(d + 1) * HW].astype(jnp.bfloat16))

            zr = jnp.zeros((1, Wp, Cin), jnp.bfloat16)
            xpad_ref[0:1] = zr
            xpad_ref[Hp - 1:Hp] = zr
            zc = jnp.zeros((Hp, 1, Cin), jnp.bfloat16)
            xpad_ref[:, 0:1] = zc
            xpad_ref[:, Wp - 1:Wp] = zc
            xpad_ref[1:H + 1, 1:W + 1] = xt.reshape(H, W, Cin)

            for ry in range(2):
                lhs = jnp.concatenate(
                    [xpad_ref[oh:oh + H, ow:ow + W, :].reshape(HW, Cin)
                     for oh in (ry, ry + 1) for ow in range(3)], axis=1)
                acc = jnp.dot(lhs, w_ref[ry],
                              preferred_element_type=jnp.float32)
                acc = acc + bias
                y = acc * pl.reciprocal(1.0 + jnp.exp(-acc), approx=True)
                for rx in range(2):
                    ot_ref[pl.ds(ry, H, 2), pl.ds(rx, W, 2), :] = (
                        y[:, rx * Cout:(rx + 1) * Cout].reshape(H, W, Cout))

            o_ref[0, :, d * 4 * HW:(d + 1) * 4 * HW] = (
                jnp.transpose(ot_ref[...].reshape(4 * HW, Cout)))

    return body


def kernel(x_ncdhw, weight, bias):
    N, Cin, D, H, W = x_ncdhw.shape
    Cout = weight.shape[1]
    HW = H * W

    x3 = x_ncdhw.reshape(N, Cin, D * HW)

    w2 = weight[:, :, 0]
    zero = jnp.zeros((Cin, Cout), w2.dtype)
    wk = jnp.stack([
        jnp.concatenate([
            jnp.concatenate([
                w2[:, :, _TAP_OF[(ry, ry + j2 // 3)], _TAP_OF[(rx, j2 % 3)]]
                if (rx, j2 % 3) in _TAP_OF else zero
                for rx in range(2)], axis=1)
            for j2 in range(6)], axis=0)
        for ry in range(2)], axis=0).astype(jnp.bfloat16)
    b2 = jnp.concatenate([bias, bias]).reshape(1, 2 * Cout).astype(jnp.float32)

    out3 = pl.pallas_call(
        _make_body(H, W, Cin, Cout, D),
        out_shape=jax.ShapeDtypeStruct((N, Cout, D * 4 * HW), x_ncdhw.dtype),
        grid=(N,),
        in_specs=[
            pl.BlockSpec((1, Cin, D * HW), lambda i: (i, 0, 0)),
            pl.BlockSpec((2, 6 * Cin, 2 * Cout), lambda i: (0, 0, 0)),
            pl.BlockSpec((1, 2 * Cout), lambda i: (0, 0)),
        ],
        out_specs=pl.BlockSpec((1, Cout, 4 * D * HW), lambda i: (i, 0, 0)),
        scratch_shapes=[
            pltpu.VMEM((H + 2, W + 2, Cin), jnp.bfloat16),
            pltpu.VMEM((2 * H, 2 * W, Cout), jnp.float32),
        ],
        compiler_params=pltpu.CompilerParams(
            dimension_semantics=("parallel",),
            vmem_limit_bytes=48 * 1024 * 1024),
    )(x3, wk, b2)

    return out3.reshape(N, Cout, D, 2 * H, 2 * W)
